# Initial kernel scaffold; baseline (speedup 1.0000x reference)
import jax
import jax.numpy as jnp
from jax.experimental import pallas as pl


def kernel(op_idx, a, b, c, op_embed, W_in, b_in, tile_keys, tile_values, tile_pos, gauge_phase, vortex_freq, W_h1, b_h1, W_h2, b_h2):
    raise NotImplementedError("write your pallas kernel here")


# speedup vs baseline: 9.3626x; 9.3626x over previous
import functools

import jax
import jax.numpy as jnp
import numpy as np
from jax.experimental import pallas as pl
from jax.experimental.pallas import tpu as pltpu

_B = 16384
_D = 128
_T = 64
_K = 4
_SPREAD = 1.5
_BLK = 2048
_NEG = -1e30


def _body(op_ref, a_ref, b_ref, c_ref, emb_ref, win_ref, bin_ref, tk_ref,
          tv_ref, tp_ref, gp_ref, vf_ref, wh1_ref, bh1_ref, wh2_ref, bh2_ref,
          res_ref, gates_ref, aux_ref, imp_acc, load_acc):
    i = pl.program_id(0)
    nb = pl.num_programs(0)
    f32 = jnp.float32

    @pl.when(i == 0)
    def _init():
        imp_acc[...] = jnp.zeros_like(imp_acc)
        load_acc[...] = jnp.zeros_like(load_acc)

    opi = op_ref[...]
    i8 = jax.lax.broadcasted_iota(jnp.int32, (1, 8), 1)
    oh8 = (opi == i8).astype(f32)
    abits = ((a_ref[...] >> i8) & 1).astype(f32)
    bbits = ((b_ref[...] >> i8) & 1).astype(f32)
    cf = c_ref[...].astype(f32)

    W = win_ref[...]
    M8 = jnp.dot(emb_ref[...], W[0:32, :], preferred_element_type=f32)
    x = (jnp.dot(oh8, M8, preferred_element_type=f32)
         + jnp.dot(abits, W[32:40, :], preferred_element_type=f32)
         + jnp.dot(bbits, W[40:48, :], preferred_element_type=f32)
         + cf * W[48:49, :]
         + bin_ref[...])

    p8 = jax.lax.broadcasted_iota(jnp.int32, (8, _T), 0).astype(f32)
    geo_tab = -((p8 - tp_ref[...]) ** 2) * (1.0 / (2.0 * _SPREAD * _SPREAD))
    scores = (jax.lax.dot_general(x, tk_ref[...], (((1,), (1,)), ((), ())),
                                  preferred_element_type=f32)
              * (1.0 / np.sqrt(_D))
              + jnp.dot(oh8, geo_tab, preferred_element_type=f32))

    lane_t = jax.lax.broadcasted_iota(jnp.int32, (1, _T), 1)
    s = scores
    vals, idxs = [], []
    for _ in range(_K):
        m = jnp.max(s, axis=1, keepdims=True)
        eq = s >= m
        idx = jnp.min(jnp.where(eq, lane_t, _T), axis=1, keepdims=True)
        vals.append(m)
        idxs.append(idx)
        s = jnp.where(lane_t == idx, _NEG, s)

    vmax = vals[0]
    es = [jnp.exp(v - vmax) for v in vals]
    den = es[0] + es[1] + es[2] + es[3]
    gs = [e / den for e in es]

    cosph = jnp.cos(gp_ref[...])
    vfreq = vf_ref[...]
    tv = tv_ref[...]
    posf = opi.astype(f32)
    lane_d = jax.lax.broadcasted_iota(jnp.int32, (1, _D), 1)
    even = (lane_d % 2) == 0
    sign = jnp.where(even, -1.0, 1.0).astype(f32)

    acc = jnp.zeros_like(x)
    load_part = jnp.zeros((1, _T), f32)
    for j in range(_K):
        oh = (lane_t == idxs[j]).astype(f32)
        v = jnp.dot(oh, tv, preferred_element_type=f32)
        gmod = jnp.sum(oh * cosph, axis=1, keepdims=True)
        fr = jnp.sum(oh * vfreq, axis=1, keepdims=True)
        theta = fr * posf
        ct = jnp.cos(theta)
        st = jnp.sin(theta)
        v = v * gmod
        vnext = pltpu.roll(v, _D - 1, 1)
        vprev = pltpu.roll(v, 1, 1)
        w = jnp.where(even, vnext, vprev)
        r = v * ct + w * (st * sign)
        acc = acc + gs[j] * r
        load_part = load_part + jnp.sum(oh * gs[j], axis=0, keepdims=True)

    out = acc + x

    sm = scores - jnp.max(scores, axis=1, keepdims=True)
    e = jnp.exp(sm)
    probs = e / jnp.sum(e, axis=1, keepdims=True)
    imp_acc[...] += jnp.sum(probs, axis=0, keepdims=True)
    load_acc[...] += load_part

    h = jnp.maximum(jnp.dot(out, wh1_ref[...], preferred_element_type=f32)
                    + bh1_ref[...], 0.0)
    z = jnp.dot(h, wh2_ref[...], preferred_element_type=f32) + bh2_ref[...]
    res_ref[...] = 1.0 / (1.0 + jnp.exp(-z))

    lane_k = jax.lax.broadcasted_iota(jnp.int32, (1, _K), 1)
    g = jnp.zeros((opi.shape[0], _K), f32)
    for j in range(_K):
        g = g + gs[j] * (lane_k == j).astype(f32)
    gates_ref[...] = g

    @pl.when(i == nb - 1)
    def _fin():
        aux_ref[...] = (_T / (float(_B) * float(_B) * _K)) * jnp.sum(
            imp_acc[...] * load_acc[...], axis=1, keepdims=True)


@functools.partial(jax.jit, static_argnames=("interpret",))
def _run(op_idx, a, b, c, op_embed, W_in, b_in, tile_keys, tile_values,
         tile_pos, gauge_phase, vortex_freq, W_h1, b_h1, W_h2, b_h2,
         interpret=False):
    B = op_idx.shape[0]
    nb = B // _BLK
    i32 = jnp.int32
    op2 = op_idx.astype(i32).reshape(B, 1)
    a2 = a.astype(i32).reshape(B, 1)
    b2 = b.astype(i32).reshape(B, 1)
    c2 = c.astype(i32).reshape(B, 1)

    row = pl.BlockSpec((_BLK, 1), lambda i: (i, 0))
    full = lambda r, co: pl.BlockSpec((r, co), lambda i: (0, 0))
    out_shapes = (
        jax.ShapeDtypeStruct((B, 8), jnp.float32),
        jax.ShapeDtypeStruct((B, _K), jnp.float32),
        jax.ShapeDtypeStruct((1, 1), jnp.float32),
    )
    res, gates, aux = pl.pallas_call(
        _body,
        grid=(nb,),
        in_specs=[
            row, row, row, row,
            full(8, 32), full(49, _D), full(1, _D), full(_T, _D),
            full(_T, _D), full(1, _T), full(1, _T), full(1, _T),
            full(_D, 64), full(1, 64), full(64, 8), full(1, 8),
        ],
        out_specs=(
            pl.BlockSpec((_BLK, 8), lambda i: (i, 0)),
            pl.BlockSpec((_BLK, _K), lambda i: (i, 0)),
            pl.BlockSpec((1, 1), lambda i: (0, 0)),
        ),
        scratch_shapes=[pltpu.VMEM((1, _T), jnp.float32),
                        pltpu.VMEM((1, _T), jnp.float32)],
        out_shape=out_shapes,
        interpret=interpret,
    )(op2, a2, b2, c2, op_embed, W_in, b_in.reshape(1, _D), tile_keys,
      tile_values, tile_pos.reshape(1, _T), gauge_phase.reshape(1, _T),
      vortex_freq.reshape(1, _T), W_h1, b_h1.reshape(1, 64), W_h2,
      b_h2.reshape(1, 8))
    return res, gates.reshape(B, 1, _K), aux.reshape(())


def kernel(op_idx, a, b, c, op_embed, W_in, b_in, tile_keys, tile_values,
           tile_pos, gauge_phase, vortex_freq, W_h1, b_h1, W_h2, b_h2):
    return _run(op_idx, a, b, c, op_embed, W_in, b_in, tile_keys, tile_values,
                tile_pos, gauge_phase, vortex_freq, W_h1, b_h1, W_h2, b_h2)
